# Initial kernel scaffold; baseline (speedup 1.0000x reference)
import jax
import jax.numpy as jnp
from jax.experimental import pallas as pl


def kernel(user_ids, item_ids, edge_index, user_table, item_table, W1, b1, W2, b2, Wp, bp):
    raise NotImplementedError("write your pallas kernel here")


# speedup vs baseline: 8.6549x; 8.6549x over previous
import functools

import jax
import jax.numpy as jnp
from jax import lax
from jax.experimental import pallas as pl
from jax.experimental.pallas import tpu as pltpu
from jax.experimental.pallas import tpu_sc as plsc

N = 50000
E = 800000
H = 32
D = 64
NC, NS, LANES = 2, 16, 16
NW = NC * NS

CH = 128
NP = 53248
PER_TILE_NODES = NP // NS
PER_WORKER_ROWS = NP // (NW * CH)
EP = 802816
EROWS = EP // CH
MSG_ROWS_PER_TILE = EROWS // NS
DEG_ROWS_PER_WORKER = EROWS // NW
TRASH = N

BLK = 2048
NBLK = NP // BLK

_mesh = plsc.VectorSubcoreMesh(core_axis_name="c", subcore_axis_name="s")
_sc_params = pltpu.CompilerParams(use_tc_tiling_on_sc=False)
f32 = jnp.float32


def _embed_deg_body(user_table, item_table, uidR, iidR, dstR, z1d, ones_h,
                    xU, xI, degp,
                    idx_v, rows_v, dst_v, ones_v, deg_sp, sem):
    c = lax.axis_index("c")
    s = lax.axis_index("s")
    wid = s * NC + c

    pltpu.sync_copy(z1d, deg_sp.at[pl.ds(s * PER_TILE_NODES, PER_TILE_NODES)])
    pltpu.sync_copy(ones_h, ones_v)
    plsc.subcore_barrier()

    @pl.loop(0, PER_WORKER_ROWS)
    def _(k):
        row = wid * PER_WORKER_ROWS + k
        pltpu.sync_copy(uidR.at[row], idx_v)
        pltpu.async_copy(user_table.at[idx_v], rows_v, sem).wait()
        pltpu.sync_copy(rows_v, xU.at[pl.ds(row * CH, CH)])
        pltpu.sync_copy(iidR.at[row], idx_v)
        pltpu.async_copy(item_table.at[idx_v], rows_v, sem).wait()
        pltpu.sync_copy(rows_v, xI.at[pl.ds(row * CH, CH)])

    @pl.loop(0, DEG_ROWS_PER_WORKER)
    def _(k):
        row = wid * DEG_ROWS_PER_WORKER + k
        pltpu.sync_copy(dstR.at[row], dst_v)
        pltpu.sync_copy(ones_v, deg_sp.at[dst_v], add=True)

    plsc.subcore_barrier()
    pltpu.sync_copy(deg_sp.at[pl.ds(s * PER_TILE_NODES, PER_TILE_NODES)],
                    degp.at[c, pl.ds(s * PER_TILE_NODES, PER_TILE_NODES)])


_embed_deg = pl.kernel(
    _embed_deg_body,
    out_type=(
        jax.ShapeDtypeStruct((NP, H), f32),
        jax.ShapeDtypeStruct((NP, H), f32),
        jax.ShapeDtypeStruct((NC, NP), f32),
    ),
    mesh=_mesh,
    scratch_types=[
        pltpu.VMEM((CH,), jnp.int32),
        pltpu.VMEM((CH, H), f32),
        pltpu.VMEM((CH,), jnp.int32),
        pltpu.VMEM((CH,), f32),
        pltpu.VMEM_SHARED((NP,), f32),
        pltpu.SemaphoreType.DMA,
    ],
    compiler_params=_sc_params,
)


def _msg_body(hsL, hsR, srcR, dstR, z2d,
              accL, accR,
              src_v, dst_v, rows_v, acc_sp, sem):
    c = lax.axis_index("c")
    s = lax.axis_index("s")

    pltpu.sync_copy(z2d, acc_sp.at[pl.ds(s * PER_TILE_NODES, PER_TILE_NODES)])
    plsc.subcore_barrier()

    def do_half(hs_ref, out_ref):
        @pl.loop(0, MSG_ROWS_PER_TILE)
        def _(k):
            row = s * MSG_ROWS_PER_TILE + k
            pltpu.sync_copy(srcR.at[row], src_v)
            pltpu.sync_copy(dstR.at[row], dst_v)
            pltpu.async_copy(hs_ref.at[src_v], rows_v, sem).wait()
            pltpu.sync_copy(rows_v, acc_sp.at[dst_v], add=True)

        plsc.subcore_barrier()
        pltpu.sync_copy(acc_sp.at[pl.ds(s * PER_TILE_NODES, PER_TILE_NODES)],
                        out_ref.at[pl.ds(s * PER_TILE_NODES, PER_TILE_NODES)])

    @pl.when(c == 0)
    def _():
        do_half(hsL, accL)

    @pl.when(c == 1)
    def _():
        do_half(hsR, accR)


_msg_pass = pl.kernel(
    _msg_body,
    out_type=(
        jax.ShapeDtypeStruct((NP, H), f32),
        jax.ShapeDtypeStruct((NP, H), f32),
    ),
    mesh=_mesh,
    scratch_types=[
        pltpu.VMEM((CH,), jnp.int32),
        pltpu.VMEM((CH,), jnp.int32),
        pltpu.VMEM((CH, H), f32),
        pltpu.VMEM_SHARED((NP, H), f32),
        pltpu.SemaphoreType.DMA,
    ],
    compiler_params=_sc_params,
)


def _tc_a_body(xu, xi, degp, w1u, w1i, hsl, hsr, dinv):
    deg = degp[0, :] + degp[1, :] + 1.0
    di = lax.rsqrt(deg)
    h = (jnp.dot(xu[...], w1u[...], preferred_element_type=f32, precision=lax.Precision.HIGHEST)
         + jnp.dot(xi[...], w1i[...], preferred_element_type=f32, precision=lax.Precision.HIGHEST))
    hs = h * di[:, None]
    hsl[...] = hs[:, :H]
    hsr[...] = hs[:, H:]
    dinv[...] = di


def _tc_mid_body(accl, accr, hsl, hsr, dinv, b, w2u, w2i, o_hsl, o_hsr):
    di = dinv[...]
    xl = jnp.maximum(di[:, None] * (accl[...] + hsl[...]) + b[0, :H], 0.0)
    xr = jnp.maximum(di[:, None] * (accr[...] + hsr[...]) + b[0, H:], 0.0)
    h = (jnp.dot(xl, w2u[...], preferred_element_type=f32, precision=lax.Precision.HIGHEST)
         + jnp.dot(xr, w2i[...], preferred_element_type=f32, precision=lax.Precision.HIGHEST))
    hs = h * di[:, None]
    o_hsl[...] = hs[:, :H]
    o_hsr[...] = hs[:, H:]


def _tc_fin_body(accl, accr, hsl, hsr, dinv, b, wp, bp, out):
    di = dinv[...]
    xl = jnp.maximum(di[:, None] * (accl[...] + hsl[...]) + b[0, :H], 0.0)
    xr = jnp.maximum(di[:, None] * (accr[...] + hsr[...]) + b[0, H:], 0.0)
    r = (jnp.sum(xl * wp[0, :H], axis=1) + jnp.sum(xr * wp[0, H:], axis=1)
         + bp[0, 0])
    out[...] = r.reshape(1, 1, BLK)


def _row_spec(width):
    return pl.BlockSpec((BLK, width), lambda i: (i, 0))


def _full_spec(shape):
    return pl.BlockSpec(shape, lambda i: tuple(0 for _ in shape))


_tc_a = pl.pallas_call(
    _tc_a_body,
    grid=(NBLK,),
    in_specs=[
        _row_spec(H), _row_spec(H),
        pl.BlockSpec((NC, BLK), lambda i: (0, i)),
        _full_spec((H, D)), _full_spec((H, D)),
    ],
    out_specs=[
        _row_spec(H), _row_spec(H),
        pl.BlockSpec((BLK,), lambda i: (i,)),
    ],
    out_shape=(
        jax.ShapeDtypeStruct((NP, H), f32),
        jax.ShapeDtypeStruct((NP, H), f32),
        jax.ShapeDtypeStruct((NP,), f32),
    ),
)

_tc_mid = pl.pallas_call(
    _tc_mid_body,
    grid=(NBLK,),
    in_specs=[
        _row_spec(H), _row_spec(H), _row_spec(H), _row_spec(H),
        pl.BlockSpec((BLK,), lambda i: (i,)),
        _full_spec((1, D)), _full_spec((H, D)), _full_spec((H, D)),
    ],
    out_specs=[_row_spec(H), _row_spec(H)],
    out_shape=(
        jax.ShapeDtypeStruct((NP, H), f32),
        jax.ShapeDtypeStruct((NP, H), f32),
    ),
)

_tc_fin = pl.pallas_call(
    _tc_fin_body,
    grid=(NBLK,),
    in_specs=[
        _row_spec(H), _row_spec(H), _row_spec(H), _row_spec(H),
        pl.BlockSpec((BLK,), lambda i: (i,)),
        _full_spec((1, D)), _full_spec((1, D)), _full_spec((1, 1)),
    ],
    out_specs=pl.BlockSpec((1, 1, BLK), lambda i: (i, 0, 0)),
    out_shape=jax.ShapeDtypeStruct((NBLK, 1, BLK), f32),
)


@jax.jit
def _run(user_ids, item_ids, edge_index, user_table, item_table,
         W1, b1, W2, b2, Wp, bp):
    uidR = jnp.pad(user_ids.astype(jnp.int32), (0, NP - N)).reshape(NP // CH, CH)
    iidR = jnp.pad(item_ids.astype(jnp.int32), (0, NP - N)).reshape(NP // CH, CH)
    src = jnp.pad(edge_index[0].astype(jnp.int32), (0, EP - E))
    dst = jnp.pad(edge_index[1].astype(jnp.int32), (0, EP - E),
                  constant_values=TRASH)
    srcR = src.reshape(EROWS, CH)
    dstR = dst.reshape(EROWS, CH)
    z1d = jnp.zeros((PER_TILE_NODES,), f32)
    z2d = jnp.zeros((PER_TILE_NODES, H), f32)
    ones_h = jnp.ones((CH,), f32)

    xU, xI, degp = _embed_deg(user_table, item_table, uidR, iidR, dstR,
                              z1d, ones_h)

    hs1L, hs1R, dinv = _tc_a(xU, xI, degp, W1[:H, :], W1[H:, :])
    acc1L, acc1R = _msg_pass(hs1L, hs1R, srcR, dstR, z2d)
    hs2L, hs2R = _tc_mid(acc1L, acc1R, hs1L, hs1R, dinv,
                         b1.reshape(1, D), W2[:H, :], W2[H:, :])
    acc2L, acc2R = _msg_pass(hs2L, hs2R, srcR, dstR, z2d)
    ratings = _tc_fin(acc2L, acc2R, hs2L, hs2R, dinv,
                      b2.reshape(1, D), Wp.reshape(1, D), bp.reshape(1, 1))
    return ratings.reshape(NP)[:N]


def kernel(user_ids, item_ids, edge_index, user_table, item_table,
           W1, b1, W2, b2, Wp, bp):
    return _run(user_ids, item_ids, edge_index, user_table, item_table,
                W1, b1, W2, b2, Wp, bp)
